# baseline (speedup 1.0000x reference)
import jax
import jax.numpy as jnp
from jax import lax
from jax.experimental import pallas as pl
from jax.experimental.pallas import tpu as pltpu
from jax.experimental.pallas import tpu_sc as plsc

N_NODES_ = 10000
N_EDGES_ = 320000
D_ = 128
TABW = 2 * D_
NCORE = 2
NSUB = 16
NWORK = NCORE * NSUB
BATCH = 96
CHB = 12
EDGES_PAD = 331776
EDGES_PER_W = EDGES_PAD // NWORK
NBATCH = EDGES_PER_W // BATCH
NCHUNK = NBATCH // CHB
STRIPE = 624
LAST0 = (NSUB - 1) * STRIPE
LASTN = N_NODES_ - LAST0


def _pack_bf16_pair(a, b):
    au = lax.bitcast_convert_type(a, jnp.uint32)
    bu = lax.bitcast_convert_type(b, jnp.uint32)
    ra = (au + 0x7FFF + ((au >> 16) & 1)) >> 16
    rb = (bu + 0x7FFF + ((bu >> 16) & 1)) >> 16
    return lax.bitcast_convert_type((rb << 16) | ra, jnp.int32)


def _mm1_body(x_ref, w_ref, o_ref):
    a = jnp.dot(x_ref[...], w_ref[0], preferred_element_type=jnp.float32)
    b = jnp.dot(x_ref[...], w_ref[1], preferred_element_type=jnp.float32)
    o_ref[...] = _pack_bf16_pair(a, b)


def _tc_build1(x, wstack):
    mb = 1000
    return pl.pallas_call(
        _mm1_body,
        grid=(N_NODES_ // mb,),
        in_specs=[
            pl.BlockSpec((mb, D_), lambda m: (m, 0)),
            pl.BlockSpec((2, D_, D_), lambda m: (0, 0, 0)),
        ],
        out_specs=pl.BlockSpec((mb, D_), lambda m: (m, 0)),
        out_shape=jax.ShapeDtypeStruct((N_NODES_, D_), jnp.int32),
    )(x, wstack)


def _mm2_body(p_ref, w_ref, o_ref):
    h = p_ref[0] + p_ref[1]
    a = jnp.dot(h, w_ref[0], preferred_element_type=jnp.float32)
    b = jnp.dot(h, w_ref[1], preferred_element_type=jnp.float32)
    o_ref[...] = _pack_bf16_pair(a, b)


def _tc_build2(hp, wstack):
    mb = 1000
    return pl.pallas_call(
        _mm2_body,
        grid=(N_NODES_ // mb,),
        in_specs=[
            pl.BlockSpec((2, mb, D_), lambda m: (0, m, 0)),
            pl.BlockSpec((2, D_, D_), lambda m: (0, 0, 0)),
        ],
        out_specs=pl.BlockSpec((mb, D_), lambda m: (m, 0)),
        out_shape=jax.ShapeDtypeStruct((N_NODES_, D_), jnp.int32),
    )(hp, wstack)


def _enc_body(x_ref, h1_ref, h2_ref, e0_ref, e1_ref, e2_ref, b_ref, o_ref):
    acc = jnp.dot(x_ref[...], e0_ref[...], preferred_element_type=jnp.float32)
    acc += jnp.dot(h1_ref[0] + h1_ref[1], e1_ref[...],
                   preferred_element_type=jnp.float32)
    acc += jnp.dot(h2_ref[0] + h2_ref[1], e2_ref[...],
                   preferred_element_type=jnp.float32)
    o_ref[...] = acc + b_ref[...]


def _tc_encode(x, h1p, h2p, e0, e1, e2, b2d, emb_dim):
    mb = 1000
    return pl.pallas_call(
        _enc_body,
        grid=(N_NODES_ // mb,),
        in_specs=[
            pl.BlockSpec((mb, D_), lambda m: (m, 0)),
            pl.BlockSpec((2, mb, D_), lambda m: (0, m, 0)),
            pl.BlockSpec((2, mb, D_), lambda m: (0, m, 0)),
            pl.BlockSpec((D_, emb_dim), lambda m: (0, 0)),
            pl.BlockSpec((D_, emb_dim), lambda m: (0, 0)),
            pl.BlockSpec((D_, emb_dim), lambda m: (0, 0)),
            pl.BlockSpec((1, emb_dim), lambda m: (0, 0)),
        ],
        out_specs=pl.BlockSpec((mb, emb_dim), lambda m: (m, 0)),
        out_shape=jax.ShapeDtypeStruct((N_NODES_, emb_dim), jnp.float32),
    )(x, h1p, h2p, e0, e1, e2, b2d)


def _edge_pass_body(g_hbm, idx_hbm, zeros_hbm,
                    out_hbm, acc, idx_v,
                    rows_a, rows_b, out_v,
                    gsem_a, gsem_b, ssem):
    c = lax.axis_index("c")
    s = lax.axis_index("s")
    wid = c * NSUB + s

    row0 = s * STRIPE

    @pl.when(s < NSUB - 1)
    def _():
        pltpu.sync_copy(zeros_hbm.at[pl.ds(row0, STRIPE)],
                        acc.at[pl.ds(row0, STRIPE)])

    @pl.when(s == NSUB - 1)
    def _():
        pltpu.sync_copy(zeros_hbm.at[pl.ds(LAST0, LASTN)],
                        acc.at[pl.ds(LAST0, LASTN)])

    plsc.subcore_barrier()

    def compute(b, rows, out):
        @pl.loop(0, BATCH // 16)
        def _(g):
            kv = idx_v[b, 2, pl.ds(g * 16, 16)]
            kws0 = lax.bitcast_convert_type(kv << 16, jnp.float32)
            kws1 = lax.bitcast_convert_type(kv & jnp.int32(-65536),
                                            jnp.float32)
            for i in range(16):
                e = g * 16 + i
                w0 = kws0[i]
                w1 = kws1[i]
                for j in range(D_ // 16):
                    v = rows[e, pl.ds(j * 16, 16)]
                    g0v = lax.bitcast_convert_type(v << 16, jnp.float32)
                    g1v = lax.bitcast_convert_type(v & jnp.int32(-65536),
                                                   jnp.float32)
                    out[e, pl.ds(j * 16, 16)] = w0 * g0v + w1 * g1v

    @pl.loop(0, NCHUNK)
    def _(ch):
        pltpu.sync_copy(idx_hbm.at[wid, ch], idx_v)

        pltpu.async_copy(g_hbm.at[idx_v.at[0, 0]], rows_a, gsem_a)
        pltpu.async_copy(g_hbm.at[idx_v.at[1, 0]], rows_b, gsem_b)

        @pl.loop(0, CHB, step=2)
        def _(b):
            pltpu.make_async_copy(g_hbm.at[idx_v.at[b, 0]], rows_a,
                                  gsem_a).wait()

            @pl.when(b >= 1)
            def _():
                pltpu.make_async_copy(out_v, acc.at[idx_v.at[b - 1, 1]],
                                      ssem).wait()

            compute(b, rows_a, out_v)
            pltpu.async_copy(out_v, acc.at[idx_v.at[b, 1]], ssem, add=True)

            @pl.when(b + 2 < CHB)
            def _():
                pltpu.async_copy(g_hbm.at[idx_v.at[b + 2, 0]], rows_a, gsem_a)

            pltpu.make_async_copy(g_hbm.at[idx_v.at[b + 1, 0]], rows_b,
                                  gsem_b).wait()

            pltpu.make_async_copy(out_v, acc.at[idx_v.at[b, 1]], ssem).wait()
            compute(b + 1, rows_b, out_v)
            pltpu.async_copy(out_v, acc.at[idx_v.at[b + 1, 1]], ssem, add=True)

            @pl.when(b + 3 < CHB)
            def _():
                pltpu.async_copy(g_hbm.at[idx_v.at[b + 3, 0]], rows_b, gsem_b)

        pltpu.make_async_copy(out_v, acc.at[idx_v.at[CHB - 1, 1]], ssem).wait()

    plsc.subcore_barrier()

    @pl.when(s < NSUB - 1)
    def _():
        pltpu.sync_copy(acc.at[pl.ds(row0, STRIPE)],
                        out_hbm.at[c, pl.ds(row0, STRIPE)])

    @pl.when(s == NSUB - 1)
    def _():
        pltpu.sync_copy(acc.at[pl.ds(LAST0, LASTN)],
                        out_hbm.at[c, pl.ds(LAST0, LASTN)])


def _sc_edge_pass(g, idxr, zeros):
    mesh = plsc.VectorSubcoreMesh(core_axis_name="c", subcore_axis_name="s",
                                  num_cores=NCORE, num_subcores=NSUB)
    kern = pl.kernel(
        _edge_pass_body,
        out_type=jax.ShapeDtypeStruct((NCORE, N_NODES_, D_), jnp.float32),
        mesh=mesh,
        scratch_types=[
            pltpu.VMEM_SHARED((N_NODES_, D_), jnp.float32),
            pltpu.VMEM((CHB, 3, BATCH), jnp.int32),
            pltpu.VMEM((BATCH, D_), jnp.int32),
            pltpu.VMEM((BATCH, D_), jnp.int32),
            pltpu.VMEM((BATCH, D_), jnp.float32),
            pltpu.SemaphoreType.DMA,
            pltpu.SemaphoreType.DMA,
            pltpu.SemaphoreType.DMA,
        ],
    )
    return kern(g, idxr, zeros)


@jax.jit
def kernel(x, edge_index, kernel_weights, W1, W2, W_enc, b_enc):
    n, d = x.shape
    emb_dim = W_enc.shape[1]

    npad = EDGES_PAD - N_EDGES_
    ei = edge_index.astype(jnp.int32)
    src = jnp.pad(ei[0], (0, npad)).reshape(NWORK, NCHUNK, CHB, BATCH)
    dst = jnp.pad(ei[1], (0, npad)).reshape(NWORK, NCHUNK, CHB, BATCH)
    kwp = _pack_bf16_pair(kernel_weights[0], kernel_weights[1])
    kwp = jnp.pad(kwp, (0, npad)).reshape(NWORK, NCHUNK, CHB, BATCH)
    idx = jnp.stack([src, dst, kwp], axis=3)
    zeros = jnp.zeros((n, d), jnp.float32)

    wstack1 = jnp.stack([W1[:d], W1[d:]])
    g1 = _tc_build1(x, wstack1)
    h1p = _sc_edge_pass(g1, idx, zeros)

    wstack2 = jnp.stack([W2[:d], W2[d:]])
    g2 = _tc_build2(h1p, wstack2)
    h2p = _sc_edge_pass(g2, idx, zeros)

    e0 = W_enc[:d]
    e1 = W_enc[d:2 * d]
    e2 = W_enc[2 * d:]
    b2d = b_enc.reshape(1, emb_dim)
    return _tc_encode(x, h1p, h2p, e0, e1, e2, b2d, emb_dim)


# speedup vs baseline: 1.0960x; 1.0960x over previous
import jax
import jax.numpy as jnp
from jax import lax
from jax.experimental import pallas as pl
from jax.experimental.pallas import tpu as pltpu
from jax.experimental.pallas import tpu_sc as plsc

N_NODES_ = 10000
N_EDGES_ = 320000
D_ = 128
TABW = 2 * D_
NCORE = 2
NSUB = 16
NWORK = NCORE * NSUB
BATCH = 64
CHB = 16
EDGES_PAD = 327680
EDGES_PER_W = EDGES_PAD // NWORK
NBATCH = EDGES_PER_W // BATCH
NCHUNK = NBATCH // CHB
STRIPE = 624
LAST0 = (NSUB - 1) * STRIPE
LASTN = N_NODES_ - LAST0


def _pack_bf16_pair(a, b):
    au = lax.bitcast_convert_type(a, jnp.uint32)
    bu = lax.bitcast_convert_type(b, jnp.uint32)
    ra = (au + 0x7FFF + ((au >> 16) & 1)) >> 16
    t = bu + 0x8000
    rb = jnp.where(t >= ra, (t - ra) >> 16, 0)
    return lax.bitcast_convert_type((rb << 16) | ra, jnp.int32)


def _mm1_body(x_ref, w_ref, o_ref):
    a = jnp.dot(x_ref[...], w_ref[0], preferred_element_type=jnp.float32)
    b = jnp.dot(x_ref[...], w_ref[1], preferred_element_type=jnp.float32)
    o_ref[...] = _pack_bf16_pair(a, b)


def _tc_build1(x, wstack):
    mb = 1000
    return pl.pallas_call(
        _mm1_body,
        grid=(N_NODES_ // mb,),
        in_specs=[
            pl.BlockSpec((mb, D_), lambda m: (m, 0)),
            pl.BlockSpec((2, D_, D_), lambda m: (0, 0, 0)),
        ],
        out_specs=pl.BlockSpec((mb, D_), lambda m: (m, 0)),
        out_shape=jax.ShapeDtypeStruct((N_NODES_, D_), jnp.int32),
    )(x, wstack)


def _mm2_body(p_ref, w_ref, o_ref):
    h = p_ref[0] + p_ref[1]
    a = jnp.dot(h, w_ref[0], preferred_element_type=jnp.float32)
    b = jnp.dot(h, w_ref[1], preferred_element_type=jnp.float32)
    o_ref[...] = _pack_bf16_pair(a, b)


def _tc_build2(hp, wstack):
    mb = 1000
    return pl.pallas_call(
        _mm2_body,
        grid=(N_NODES_ // mb,),
        in_specs=[
            pl.BlockSpec((2, mb, D_), lambda m: (0, m, 0)),
            pl.BlockSpec((2, D_, D_), lambda m: (0, 0, 0)),
        ],
        out_specs=pl.BlockSpec((mb, D_), lambda m: (m, 0)),
        out_shape=jax.ShapeDtypeStruct((N_NODES_, D_), jnp.int32),
    )(hp, wstack)


def _enc_body(x_ref, h1_ref, h2_ref, e0_ref, e1_ref, e2_ref, b_ref, o_ref):
    acc = jnp.dot(x_ref[...], e0_ref[...], preferred_element_type=jnp.float32)
    acc += jnp.dot(h1_ref[0] + h1_ref[1], e1_ref[...],
                   preferred_element_type=jnp.float32)
    acc += jnp.dot(h2_ref[0] + h2_ref[1], e2_ref[...],
                   preferred_element_type=jnp.float32)
    o_ref[...] = acc + b_ref[...]


def _tc_encode(x, h1p, h2p, e0, e1, e2, b2d, emb_dim):
    mb = 1000
    return pl.pallas_call(
        _enc_body,
        grid=(N_NODES_ // mb,),
        in_specs=[
            pl.BlockSpec((mb, D_), lambda m: (m, 0)),
            pl.BlockSpec((2, mb, D_), lambda m: (0, m, 0)),
            pl.BlockSpec((2, mb, D_), lambda m: (0, m, 0)),
            pl.BlockSpec((D_, emb_dim), lambda m: (0, 0)),
            pl.BlockSpec((D_, emb_dim), lambda m: (0, 0)),
            pl.BlockSpec((D_, emb_dim), lambda m: (0, 0)),
            pl.BlockSpec((1, emb_dim), lambda m: (0, 0)),
        ],
        out_specs=pl.BlockSpec((mb, emb_dim), lambda m: (m, 0)),
        out_shape=jax.ShapeDtypeStruct((N_NODES_, emb_dim), jnp.float32),
    )(x, h1p, h2p, e0, e1, e2, b2d)


def _edge_pass_body(g_hbm, src_hbm, dst_hbm, kw0_hbm, kw1_hbm, zeros_hbm,
                    out_hbm, acc, src_v, dst_v, kw0_v, kw1_v,
                    rows_a, rows_b, out_a, out_b,
                    gsem_a, gsem_b, ssem_a, ssem_b):
    c = lax.axis_index("c")
    s = lax.axis_index("s")
    wid = c * NSUB + s

    row0 = s * STRIPE

    @pl.when(s < NSUB - 1)
    def _():
        pltpu.sync_copy(zeros_hbm.at[pl.ds(row0, STRIPE)],
                        acc.at[pl.ds(row0, STRIPE)])

    @pl.when(s == NSUB - 1)
    def _():
        pltpu.sync_copy(zeros_hbm.at[pl.ds(LAST0, LASTN)],
                        acc.at[pl.ds(LAST0, LASTN)])

    plsc.subcore_barrier()

    def compute(b, rows, out):
        @pl.loop(0, BATCH // 16)
        def _(g):
            kws0 = kw0_v[pl.ds(b * BATCH + g * 16, 16)]
            kws1 = kw1_v[pl.ds(b * BATCH + g * 16, 16)]
            for i in range(16):
                e = g * 16 + i
                w0 = kws0[i]
                w1 = kws1[i]
                for j in range(D_ // 16):
                    v = rows[e, pl.ds(j * 16, 16)]
                    g0v = lax.bitcast_convert_type(v << 16, jnp.float32)
                    g1v = lax.bitcast_convert_type(v, jnp.float32)
                    out[e, pl.ds(j * 16, 16)] = w0 * g0v + w1 * g1v

    @pl.loop(0, NCHUNK)
    def _(ch):
        pltpu.sync_copy(src_hbm.at[wid, ch], src_v)
        pltpu.sync_copy(dst_hbm.at[wid, ch], dst_v)
        pltpu.sync_copy(kw0_hbm.at[wid, ch], kw0_v)
        pltpu.sync_copy(kw1_hbm.at[wid, ch], kw1_v)

        pltpu.async_copy(g_hbm.at[src_v.at[0]], rows_a, gsem_a)
        pltpu.async_copy(g_hbm.at[src_v.at[1]], rows_b, gsem_b)

        @pl.loop(0, CHB, step=2)
        def _(b):
            pltpu.make_async_copy(g_hbm.at[src_v.at[b]], rows_a, gsem_a).wait()

            @pl.when(b >= 2)
            def _():
                pltpu.make_async_copy(out_a, acc.at[dst_v.at[b - 2]],
                                      ssem_a).wait()

            compute(b, rows_a, out_a)
            pltpu.async_copy(out_a, acc.at[dst_v.at[b]], ssem_a, add=True)

            @pl.when(b + 2 < CHB)
            def _():
                pltpu.async_copy(g_hbm.at[src_v.at[b + 2]], rows_a, gsem_a)

            pltpu.make_async_copy(g_hbm.at[src_v.at[b + 1]], rows_b,
                                  gsem_b).wait()

            @pl.when(b >= 2)
            def _():
                pltpu.make_async_copy(out_b, acc.at[dst_v.at[b - 1]],
                                      ssem_b).wait()

            compute(b + 1, rows_b, out_b)
            pltpu.async_copy(out_b, acc.at[dst_v.at[b + 1]], ssem_b, add=True)

            @pl.when(b + 3 < CHB)
            def _():
                pltpu.async_copy(g_hbm.at[src_v.at[b + 3]], rows_b, gsem_b)

        pltpu.make_async_copy(out_a, acc.at[dst_v.at[CHB - 2]], ssem_a).wait()
        pltpu.make_async_copy(out_b, acc.at[dst_v.at[CHB - 1]], ssem_b).wait()

    plsc.subcore_barrier()

    @pl.when(s < NSUB - 1)
    def _():
        pltpu.sync_copy(acc.at[pl.ds(row0, STRIPE)],
                        out_hbm.at[c, pl.ds(row0, STRIPE)])

    @pl.when(s == NSUB - 1)
    def _():
        pltpu.sync_copy(acc.at[pl.ds(LAST0, LASTN)],
                        out_hbm.at[c, pl.ds(LAST0, LASTN)])


def _sc_edge_pass(g, srcr, dstr, kw0r, kw1r, zeros):
    mesh = plsc.VectorSubcoreMesh(core_axis_name="c", subcore_axis_name="s",
                                  num_cores=NCORE, num_subcores=NSUB)
    kern = pl.kernel(
        _edge_pass_body,
        out_type=jax.ShapeDtypeStruct((NCORE, N_NODES_, D_), jnp.float32),
        mesh=mesh,
        scratch_types=[
            pltpu.VMEM_SHARED((N_NODES_, D_), jnp.float32),
            pltpu.VMEM((CHB, BATCH), jnp.int32),
            pltpu.VMEM((CHB, BATCH), jnp.int32),
            pltpu.VMEM((CHB * BATCH,), jnp.float32),
            pltpu.VMEM((CHB * BATCH,), jnp.float32),
            pltpu.VMEM((BATCH, D_), jnp.int32),
            pltpu.VMEM((BATCH, D_), jnp.int32),
            pltpu.VMEM((BATCH, D_), jnp.float32),
            pltpu.VMEM((BATCH, D_), jnp.float32),
            pltpu.SemaphoreType.DMA,
            pltpu.SemaphoreType.DMA,
            pltpu.SemaphoreType.DMA,
            pltpu.SemaphoreType.DMA,
        ],
    )
    return kern(g, srcr, dstr, kw0r, kw1r, zeros)


@jax.jit
def kernel(x, edge_index, kernel_weights, W1, W2, W_enc, b_enc):
    n, d = x.shape
    emb_dim = W_enc.shape[1]

    npad = EDGES_PAD - N_EDGES_
    ei = edge_index.astype(jnp.int32)
    src = jnp.pad(ei[0], (0, npad)).reshape(NWORK, NCHUNK, CHB, BATCH)
    dst = jnp.pad(ei[1], (0, npad)).reshape(NWORK, NCHUNK, CHB, BATCH)
    kw0 = jnp.pad(kernel_weights[0], (0, npad)).reshape(NWORK, NCHUNK,
                                                        CHB * BATCH)
    kw1 = jnp.pad(kernel_weights[1], (0, npad)).reshape(NWORK, NCHUNK,
                                                        CHB * BATCH)
    zeros = jnp.zeros((n, d), jnp.float32)

    wstack1 = jnp.stack([W1[:d], W1[d:]])
    g1 = _tc_build1(x, wstack1)
    h1p = _sc_edge_pass(g1, src, dst, kw0, kw1, zeros)

    wstack2 = jnp.stack([W2[:d], W2[d:]])
    g2 = _tc_build2(h1p, wstack2)
    h2p = _sc_edge_pass(g2, src, dst, kw0, kw1, zeros)

    e0 = W_enc[:d]
    e1 = W_enc[d:2 * d]
    e2 = W_enc[2 * d:]
    b2d = b_enc.reshape(1, emb_dim)
    return _tc_encode(x, h1p, h2p, e0, e1, e2, b2d, emb_dim)


# speedup vs baseline: 1.1226x; 1.0243x over previous
import jax
import jax.numpy as jnp
from jax import lax
from jax.experimental import pallas as pl
from jax.experimental.pallas import tpu as pltpu
from jax.experimental.pallas import tpu_sc as plsc

N_NODES_ = 10000
N_EDGES_ = 320000
D_ = 128
TABW = 2 * D_
NCORE = 2
NSUB = 16
NWORK = NCORE * NSUB
BATCH = 64
CHB = 16
EDGES_PAD = 327680
NCHUNK0 = 12
NCHUNK1 = 8
NCHUNKT = NCHUNK0 + NCHUNK1
STRIPE = 624
LAST0 = (NSUB - 1) * STRIPE
LASTN = N_NODES_ - LAST0


def _pack_bf16_pair(a, b):
    au = lax.bitcast_convert_type(a, jnp.uint32)
    bu = lax.bitcast_convert_type(b, jnp.uint32)
    ra = (au + 0x7FFF + ((au >> 16) & 1)) >> 16
    t = bu + 0x8000
    rb = jnp.where(t >= ra, (t - ra) >> 16, 0)
    return lax.bitcast_convert_type((rb << 16) | ra, jnp.int32)


def _mm1_body(x_ref, w_ref, o_ref):
    a = jnp.dot(x_ref[...], w_ref[0], preferred_element_type=jnp.float32)
    b = jnp.dot(x_ref[...], w_ref[1], preferred_element_type=jnp.float32)
    o_ref[...] = _pack_bf16_pair(a, b)


def _tc_build1(x, wstack):
    mb = 1000
    return pl.pallas_call(
        _mm1_body,
        grid=(N_NODES_ // mb,),
        in_specs=[
            pl.BlockSpec((mb, D_), lambda m: (m, 0)),
            pl.BlockSpec((2, D_, D_), lambda m: (0, 0, 0)),
        ],
        out_specs=pl.BlockSpec((mb, D_), lambda m: (m, 0)),
        out_shape=jax.ShapeDtypeStruct((N_NODES_, D_), jnp.int32),
    )(x, wstack)


def _mm2_body(p_ref, w_ref, o_ref):
    h = p_ref[0] + p_ref[1]
    a = jnp.dot(h, w_ref[0], preferred_element_type=jnp.float32)
    b = jnp.dot(h, w_ref[1], preferred_element_type=jnp.float32)
    o_ref[...] = _pack_bf16_pair(a, b)


def _tc_build2(hp, wstack):
    mb = 1000
    return pl.pallas_call(
        _mm2_body,
        grid=(N_NODES_ // mb,),
        in_specs=[
            pl.BlockSpec((2, mb, D_), lambda m: (0, m, 0)),
            pl.BlockSpec((2, D_, D_), lambda m: (0, 0, 0)),
        ],
        out_specs=pl.BlockSpec((mb, D_), lambda m: (m, 0)),
        out_shape=jax.ShapeDtypeStruct((N_NODES_, D_), jnp.int32),
    )(hp, wstack)


def _enc_body(x_ref, h1_ref, h2_ref, e0_ref, e1_ref, e2_ref, b_ref, o_ref):
    acc = jnp.dot(x_ref[...], e0_ref[...], preferred_element_type=jnp.float32)
    acc += jnp.dot(h1_ref[0] + h1_ref[1], e1_ref[...],
                   preferred_element_type=jnp.float32)
    acc += jnp.dot(h2_ref[0] + h2_ref[1], e2_ref[...],
                   preferred_element_type=jnp.float32)
    o_ref[...] = acc + b_ref[...]


def _tc_encode(x, h1p, h2p, e0, e1, e2, b2d, emb_dim):
    mb = 1000
    return pl.pallas_call(
        _enc_body,
        grid=(N_NODES_ // mb,),
        in_specs=[
            pl.BlockSpec((mb, D_), lambda m: (m, 0)),
            pl.BlockSpec((2, mb, D_), lambda m: (0, m, 0)),
            pl.BlockSpec((2, mb, D_), lambda m: (0, m, 0)),
            pl.BlockSpec((D_, emb_dim), lambda m: (0, 0)),
            pl.BlockSpec((D_, emb_dim), lambda m: (0, 0)),
            pl.BlockSpec((D_, emb_dim), lambda m: (0, 0)),
            pl.BlockSpec((1, emb_dim), lambda m: (0, 0)),
        ],
        out_specs=pl.BlockSpec((mb, emb_dim), lambda m: (m, 0)),
        out_shape=jax.ShapeDtypeStruct((N_NODES_, emb_dim), jnp.float32),
    )(x, h1p, h2p, e0, e1, e2, b2d)


def _edge_pass_body(g_hbm, src_hbm, dst_hbm, kw0_hbm, kw1_hbm, zeros_hbm,
                    out_hbm, acc, src_v, dst_v, kw0_v, kw1_v,
                    rows_a, rows_b, out_a, out_b,
                    gsem_a, gsem_b, ssem_a, ssem_b):
    c = lax.axis_index("c")
    s = lax.axis_index("s")
    base = c * NCHUNK0

    row0 = s * STRIPE

    @pl.when(s < NSUB - 1)
    def _():
        pltpu.sync_copy(zeros_hbm.at[pl.ds(row0, STRIPE)],
                        acc.at[pl.ds(row0, STRIPE)])

    @pl.when(s == NSUB - 1)
    def _():
        pltpu.sync_copy(zeros_hbm.at[pl.ds(LAST0, LASTN)],
                        acc.at[pl.ds(LAST0, LASTN)])

    plsc.subcore_barrier()

    def compute(b, rows, out):
        @pl.loop(0, BATCH // 16)
        def _(g):
            kws0 = kw0_v[pl.ds(b * BATCH + g * 16, 16)]
            kws1 = kw1_v[pl.ds(b * BATCH + g * 16, 16)]
            for i in range(16):
                e = g * 16 + i
                w0 = kws0[i]
                w1 = kws1[i]
                for j in range(D_ // 16):
                    v = rows[e, pl.ds(j * 16, 16)]
                    g0v = lax.bitcast_convert_type(v << 16, jnp.float32)
                    g1v = lax.bitcast_convert_type(v, jnp.float32)
                    out[e, pl.ds(j * 16, 16)] = w0 * g0v + w1 * g1v

    def run_chunks(nch):
        @pl.loop(0, nch)
        def _(ch):
            chunk_body(ch)

    def chunk_body(ch):
        pltpu.sync_copy(src_hbm.at[s, base + ch], src_v)
        pltpu.sync_copy(dst_hbm.at[s, base + ch], dst_v)
        pltpu.sync_copy(kw0_hbm.at[s, base + ch], kw0_v)
        pltpu.sync_copy(kw1_hbm.at[s, base + ch], kw1_v)

        pltpu.async_copy(g_hbm.at[src_v.at[0]], rows_a, gsem_a)
        pltpu.async_copy(g_hbm.at[src_v.at[1]], rows_b, gsem_b)

        @pl.loop(0, CHB, step=2)
        def _(b):
            pltpu.make_async_copy(g_hbm.at[src_v.at[b]], rows_a, gsem_a).wait()

            @pl.when(b >= 2)
            def _():
                pltpu.make_async_copy(out_a, acc.at[dst_v.at[b - 2]],
                                      ssem_a).wait()

            compute(b, rows_a, out_a)
            pltpu.async_copy(out_a, acc.at[dst_v.at[b]], ssem_a, add=True)

            @pl.when(b + 2 < CHB)
            def _():
                pltpu.async_copy(g_hbm.at[src_v.at[b + 2]], rows_a, gsem_a)

            pltpu.make_async_copy(g_hbm.at[src_v.at[b + 1]], rows_b,
                                  gsem_b).wait()

            @pl.when(b >= 2)
            def _():
                pltpu.make_async_copy(out_b, acc.at[dst_v.at[b - 1]],
                                      ssem_b).wait()

            compute(b + 1, rows_b, out_b)
            pltpu.async_copy(out_b, acc.at[dst_v.at[b + 1]], ssem_b, add=True)

            @pl.when(b + 3 < CHB)
            def _():
                pltpu.async_copy(g_hbm.at[src_v.at[b + 3]], rows_b, gsem_b)

        pltpu.make_async_copy(out_a, acc.at[dst_v.at[CHB - 2]], ssem_a).wait()
        pltpu.make_async_copy(out_b, acc.at[dst_v.at[CHB - 1]], ssem_b).wait()

    @pl.when(c == 0)
    def _():
        run_chunks(NCHUNK0)

    @pl.when(c == 1)
    def _():
        run_chunks(NCHUNK1)

    plsc.subcore_barrier()

    @pl.when(s < NSUB - 1)
    def _():
        pltpu.sync_copy(acc.at[pl.ds(row0, STRIPE)],
                        out_hbm.at[c, pl.ds(row0, STRIPE)])

    @pl.when(s == NSUB - 1)
    def _():
        pltpu.sync_copy(acc.at[pl.ds(LAST0, LASTN)],
                        out_hbm.at[c, pl.ds(LAST0, LASTN)])


def _sc_edge_pass(g, srcr, dstr, kw0r, kw1r, zeros):
    mesh = plsc.VectorSubcoreMesh(core_axis_name="c", subcore_axis_name="s",
                                  num_cores=NCORE, num_subcores=NSUB)
    kern = pl.kernel(
        _edge_pass_body,
        out_type=jax.ShapeDtypeStruct((NCORE, N_NODES_, D_), jnp.float32),
        mesh=mesh,
        scratch_types=[
            pltpu.VMEM_SHARED((N_NODES_, D_), jnp.float32),
            pltpu.VMEM((CHB, BATCH), jnp.int32),
            pltpu.VMEM((CHB, BATCH), jnp.int32),
            pltpu.VMEM((CHB * BATCH,), jnp.float32),
            pltpu.VMEM((CHB * BATCH,), jnp.float32),
            pltpu.VMEM((BATCH, D_), jnp.int32),
            pltpu.VMEM((BATCH, D_), jnp.int32),
            pltpu.VMEM((BATCH, D_), jnp.float32),
            pltpu.VMEM((BATCH, D_), jnp.float32),
            pltpu.SemaphoreType.DMA,
            pltpu.SemaphoreType.DMA,
            pltpu.SemaphoreType.DMA,
            pltpu.SemaphoreType.DMA,
        ],
    )
    return kern(g, srcr, dstr, kw0r, kw1r, zeros)


@jax.jit
def kernel(x, edge_index, kernel_weights, W1, W2, W_enc, b_enc):
    n, d = x.shape
    emb_dim = W_enc.shape[1]

    npad = EDGES_PAD - N_EDGES_
    ei = edge_index.astype(jnp.int32)
    src = jnp.pad(ei[0], (0, npad)).reshape(NSUB, NCHUNKT, CHB, BATCH)
    dst = jnp.pad(ei[1], (0, npad)).reshape(NSUB, NCHUNKT, CHB, BATCH)
    kw0 = jnp.pad(kernel_weights[0], (0, npad)).reshape(NSUB, NCHUNKT,
                                                        CHB * BATCH)
    kw1 = jnp.pad(kernel_weights[1], (0, npad)).reshape(NSUB, NCHUNKT,
                                                        CHB * BATCH)
    zeros = jnp.zeros((n, d), jnp.float32)

    wstack1 = jnp.stack([W1[:d], W1[d:]])
    g1 = _tc_build1(x, wstack1)
    h1p = _sc_edge_pass(g1, src, dst, kw0, kw1, zeros)

    wstack2 = jnp.stack([W2[:d], W2[d:]])
    g2 = _tc_build2(h1p, wstack2)
    h2p = _sc_edge_pass(g2, src, dst, kw0, kw1, zeros)

    e0 = W_enc[:d]
    e1 = W_enc[d:2 * d]
    e2 = W_enc[2 * d:]
    b2d = b_enc.reshape(1, emb_dim)
    return _tc_encode(x, h1p, h2p, e0, e1, e2, b2d, emb_dim)
